# Initial kernel scaffold; baseline (speedup 1.0000x reference)
import jax
import jax.numpy as jnp
from jax.experimental import pallas as pl


def kernel(node_feats, edge_feats, expanded_dists, edge_index, W1, b1, W2, b2, We, be):
    raise NotImplementedError("write your pallas kernel here")


# speedup vs baseline: 1.6825x; 1.6825x over previous
import functools

import jax
import jax.numpy as jnp
from jax import lax
from jax.experimental import pallas as pl
from jax.experimental.pallas import tpu as pltpu
from jax.experimental.pallas import tpu_sc as plsc

N = 10000
E = 160000
F = 256
D = 128
FH = F // 2


_BE = 1000


def _tc_body(ed_ref, ef_ref, w1_ref, b1_ref, w2_ref, b2_ref, we_ref, be_ref,
             dist_ref, he_ref):
    x = jnp.dot(ed_ref[...], w1_ref[...], preferred_element_type=jnp.float32)
    x = x + b1_ref[...]
    h = jnp.where(x * 0.5 > 14.0, x, 2.0 * jnp.log1p(jnp.exp(0.5 * x)))
    dist = jnp.dot(h, w2_ref[...], preferred_element_type=jnp.float32) + b2_ref[...]
    dist_ref[0] = dist[:, :FH]
    dist_ref[1] = dist[:, FH:]
    he_ref[...] = jnp.dot(ef_ref[...], we_ref[...],
                          preferred_element_type=jnp.float32) + be_ref[...]


def _tc_dense(ed, ef, W1, b1, W2, b2, We, be):
    grid = (E // _BE,)
    full = lambda shape: pl.BlockSpec(shape, lambda i: (0,) * len(shape))
    return pl.pallas_call(
        _tc_body,
        grid=grid,
        in_specs=[
            pl.BlockSpec((_BE, D), lambda i: (i, 0)),
            pl.BlockSpec((_BE, F), lambda i: (i, 0)),
            full((D, F)), full((1, F)), full((F, F)), full((1, F)),
            full((F, F)), full((1, F)),
        ],
        out_specs=[
            pl.BlockSpec((2, _BE, FH), lambda i: (0, i, 0)),
            pl.BlockSpec((_BE, F), lambda i: (i, 0)),
        ],
        out_shape=[
            jax.ShapeDtypeStruct((2, E, FH), jnp.float32),
            jax.ShapeDtypeStruct((E, F), jnp.float32),
        ],
    )(ed, ef, W1, b1.reshape(1, F), W2, b2.reshape(1, F), We, be.reshape(1, F))


_C = 80
_EPT = E // 16
_NCH = _EPT // _C
_NPT = N // 16


def _sc_body(node_hbm, dist_hbm, he_hbm, src_hbm, dst_hbm, out_hbm,
             acc, srcv, gidx, dstv, nfv, distv, hev, msgv, sem):
    c = lax.axis_index("c")
    s = lax.axis_index("s")

    def _zrow(i, _):
        for j in range(FH // 16):
            msgv[i, pl.ds(j * 16, 16)] = jnp.zeros((16,), jnp.float32)
        return ()
    lax.fori_loop(0, _C, _zrow, ())
    base = s * _NPT
    for k in range(_NPT // _C):
        pltpu.sync_copy(msgv, acc.at[pl.ds(base + k * _C, _C)])
    rem = _NPT % _C
    if rem:
        pltpu.sync_copy(msgv.at[pl.ds(0, rem)],
                        acc.at[pl.ds(base + (_NPT // _C) * _C, rem)])
    plsc.subcore_barrier()

    tile_base = s * _EPT

    def _chunk(k, _):
        e0 = tile_base + k * _C
        pltpu.sync_copy(src_hbm.at[pl.ds(e0, _C)], srcv)
        pltpu.sync_copy(dst_hbm.at[pl.ds(e0, _C)], dstv)
        for g in range(_C // 16):
            sl = pl.ds(g * 16, 16)
            gidx[sl] = srcv[sl] * 2 + c
        pltpu.async_copy(node_hbm.at[gidx], nfv, sem).wait()
        pltpu.sync_copy(dist_hbm.at[c, pl.ds(e0, _C)], distv)
        pltpu.sync_copy(he_hbm.at[pl.ds(e0, _C), c], hev)

        def _fma(i, _):
            for j in range(FH // 16):
                sl = pl.ds(j * 16, 16)
                msgv[i, sl] = nfv[i, sl] * distv[i, sl] + hev[i, sl]
            return ()
        lax.fori_loop(0, _C, _fma, ())
        pltpu.sync_copy(msgv, acc.at[dstv], add=True)
        return ()

    lax.fori_loop(0, _NCH, _chunk, ())
    plsc.subcore_barrier()

    r0 = s * _NPT
    pltpu.sync_copy(acc.at[pl.ds(r0, _NPT)], out_hbm.at[pl.ds(r0, _NPT), c])


def _sc_message(node_flat, dist_split, he, src, dst):
    mesh = plsc.VectorSubcoreMesh(core_axis_name="c", subcore_axis_name="s")
    f = pl.kernel(
        _sc_body,
        out_type=jax.ShapeDtypeStruct((N, 2, FH), jnp.float32),
        mesh=mesh,
        scratch_types=[
            pltpu.VMEM_SHARED((N, FH), jnp.float32),
            pltpu.VMEM((_C,), jnp.int32),
            pltpu.VMEM((_C,), jnp.int32),
            pltpu.VMEM((_C,), jnp.int32),
            pltpu.VMEM((_C, FH), jnp.float32),
            pltpu.VMEM((_C, FH), jnp.float32),
            pltpu.VMEM((_C, FH), jnp.float32),
            pltpu.VMEM((_C, FH), jnp.float32),
            pltpu.SemaphoreType.DMA,
        ],
    )
    return f(node_flat, dist_split, he.reshape(E, 2, FH), src, dst)


def kernel(node_feats, edge_feats, expanded_dists, edge_index, W1, b1, W2, b2, We, be):
    dist_split, he = _tc_dense(expanded_dists, edge_feats, W1, b1, W2, b2, We, be)
    src = edge_index[0]
    dst = edge_index[1]
    node_flat = node_feats.reshape(2 * N, FH)
    agg = _sc_message(node_flat, dist_split, he, src, dst)
    return (agg.reshape(N, F), he)


# speedup vs baseline: 1.6835x; 1.0006x over previous
import functools

import jax
import jax.numpy as jnp
from jax import lax
from jax.experimental import pallas as pl
from jax.experimental.pallas import tpu as pltpu
from jax.experimental.pallas import tpu_sc as plsc

N = 10000
E = 160000
F = 256
D = 128
FH = F // 2


_BE = 1000


def _tc_body(ed_ref, ef_ref, w1_ref, b1_ref, w2_ref, b2_ref, we_ref, be_ref,
             dist_ref, he_ref):
    bf = jnp.bfloat16
    x = jnp.dot(ed_ref[...].astype(bf), w1_ref[...].astype(bf),
                preferred_element_type=jnp.float32)
    x = x + b1_ref[...]
    h = jnp.where(x * 0.5 > 14.0, x, 2.0 * jnp.log1p(jnp.exp(0.5 * x)))
    dist = jnp.dot(h.astype(bf), w2_ref[...].astype(bf),
                   preferred_element_type=jnp.float32) + b2_ref[...]
    dist_ref[0] = dist[:, :FH]
    dist_ref[1] = dist[:, FH:]
    he_ref[...] = jnp.dot(ef_ref[...].astype(bf), we_ref[...].astype(bf),
                          preferred_element_type=jnp.float32) + be_ref[...]


def _tc_dense(ed, ef, W1, b1, W2, b2, We, be):
    grid = (E // _BE,)
    full = lambda shape: pl.BlockSpec(shape, lambda i: (0,) * len(shape))
    return pl.pallas_call(
        _tc_body,
        grid=grid,
        in_specs=[
            pl.BlockSpec((_BE, D), lambda i: (i, 0)),
            pl.BlockSpec((_BE, F), lambda i: (i, 0)),
            full((D, F)), full((1, F)), full((F, F)), full((1, F)),
            full((F, F)), full((1, F)),
        ],
        out_specs=[
            pl.BlockSpec((2, _BE, FH), lambda i: (0, i, 0)),
            pl.BlockSpec((_BE, F), lambda i: (i, 0)),
        ],
        out_shape=[
            jax.ShapeDtypeStruct((2, E, FH), jnp.float32),
            jax.ShapeDtypeStruct((E, F), jnp.float32),
        ],
    )(ed, ef, W1, b1.reshape(1, F), W2, b2.reshape(1, F), We, be.reshape(1, F))


_C = 80
_EPT = E // 16
_NCH = _EPT // _C
_NPT = N // 16


def _sc_body(node_hbm, dist_hbm, he_hbm, src_hbm, dst_hbm, out_hbm,
             acc, srcv, gidx, dstv, nfv, distv, hev, msgv, sem):
    c = lax.axis_index("c")
    s = lax.axis_index("s")

    def _zrow(i, _):
        for j in range(FH // 16):
            msgv[i, pl.ds(j * 16, 16)] = jnp.zeros((16,), jnp.float32)
        return ()
    lax.fori_loop(0, _C, _zrow, ())
    base = s * _NPT
    for k in range(_NPT // _C):
        pltpu.sync_copy(msgv, acc.at[pl.ds(base + k * _C, _C)])
    rem = _NPT % _C
    if rem:
        pltpu.sync_copy(msgv.at[pl.ds(0, rem)],
                        acc.at[pl.ds(base + (_NPT // _C) * _C, rem)])
    plsc.subcore_barrier()

    tile_base = s * _EPT

    def _chunk(k, _):
        e0 = tile_base + k * _C
        pltpu.sync_copy(src_hbm.at[pl.ds(e0, _C)], srcv)
        pltpu.sync_copy(dst_hbm.at[pl.ds(e0, _C)], dstv)
        for g in range(_C // 16):
            sl = pl.ds(g * 16, 16)
            gidx[sl] = srcv[sl] * 2 + c
        pltpu.async_copy(node_hbm.at[gidx], nfv, sem).wait()
        pltpu.sync_copy(dist_hbm.at[c, pl.ds(e0, _C)], distv)
        pltpu.sync_copy(he_hbm.at[pl.ds(e0, _C), c], hev)

        def _fma(i, _):
            for j in range(FH // 16):
                sl = pl.ds(j * 16, 16)
                msgv[i, sl] = nfv[i, sl] * distv[i, sl] + hev[i, sl]
            return ()
        lax.fori_loop(0, _C, _fma, ())
        pltpu.sync_copy(msgv, acc.at[dstv], add=True)
        return ()

    lax.fori_loop(0, _NCH, _chunk, ())
    plsc.subcore_barrier()

    r0 = s * _NPT
    pltpu.sync_copy(acc.at[pl.ds(r0, _NPT)], out_hbm.at[pl.ds(r0, _NPT), c])


def _sc_message(node_flat, dist_split, he, src, dst):
    mesh = plsc.VectorSubcoreMesh(core_axis_name="c", subcore_axis_name="s")
    f = pl.kernel(
        _sc_body,
        out_type=jax.ShapeDtypeStruct((N, 2, FH), jnp.float32),
        mesh=mesh,
        scratch_types=[
            pltpu.VMEM_SHARED((N, FH), jnp.float32),
            pltpu.VMEM((_C,), jnp.int32),
            pltpu.VMEM((_C,), jnp.int32),
            pltpu.VMEM((_C,), jnp.int32),
            pltpu.VMEM((_C, FH), jnp.float32),
            pltpu.VMEM((_C, FH), jnp.float32),
            pltpu.VMEM((_C, FH), jnp.float32),
            pltpu.VMEM((_C, FH), jnp.float32),
            pltpu.SemaphoreType.DMA,
        ],
    )
    return f(node_flat, dist_split, he.reshape(E, 2, FH), src, dst)


def kernel(node_feats, edge_feats, expanded_dists, edge_index, W1, b1, W2, b2, We, be):
    dist_split, he = _tc_dense(expanded_dists, edge_feats, W1, b1, W2, b2, We, be)
    src = edge_index[0]
    dst = edge_index[1]
    node_flat = node_feats.reshape(2 * N, FH)
    agg = _sc_message(node_flat, dist_split, he, src, dst)
    return (agg.reshape(N, F), he)


# speedup vs baseline: 2.2615x; 1.3434x over previous
import functools

import jax
import jax.numpy as jnp
from jax import lax
from jax.experimental import pallas as pl
from jax.experimental.pallas import tpu as pltpu
from jax.experimental.pallas import tpu_sc as plsc

N = 10000
E = 160000
F = 256
D = 128
FH = F // 2


_BE = 1000


def _tc_body(ed_ref, ef_ref, w1_ref, b1_ref, w2_ref, b2_ref, we_ref, be_ref,
             dist_ref, he_ref):
    bf = jnp.bfloat16
    x = jnp.dot(ed_ref[...].astype(bf), w1_ref[...].astype(bf),
                preferred_element_type=jnp.float32)
    x = x + b1_ref[...]
    h = jnp.where(x * 0.5 > 14.0, x, 2.0 * jnp.log1p(jnp.exp(0.5 * x)))
    dist = jnp.dot(h.astype(bf), w2_ref[...].astype(bf),
                   preferred_element_type=jnp.float32) + b2_ref[...]
    dist_ref[0] = dist[:, :FH]
    dist_ref[1] = dist[:, FH:]
    he_ref[...] = jnp.dot(ef_ref[...].astype(bf), we_ref[...].astype(bf),
                          preferred_element_type=jnp.float32) + be_ref[...]


def _tc_dense(ed, ef, W1, b1, W2, b2, We, be):
    grid = (E // _BE,)
    full = lambda shape: pl.BlockSpec(shape, lambda i: (0,) * len(shape))
    return pl.pallas_call(
        _tc_body,
        grid=grid,
        in_specs=[
            pl.BlockSpec((_BE, D), lambda i: (i, 0)),
            pl.BlockSpec((_BE, F), lambda i: (i, 0)),
            full((D, F)), full((1, F)), full((F, F)), full((1, F)),
            full((F, F)), full((1, F)),
        ],
        out_specs=[
            pl.BlockSpec((2, _BE, FH), lambda i: (0, i, 0)),
            pl.BlockSpec((_BE, F), lambda i: (i, 0)),
        ],
        out_shape=[
            jax.ShapeDtypeStruct((2, E, FH), jnp.float32),
            jax.ShapeDtypeStruct((E, F), jnp.float32),
        ],
    )(ed, ef, W1, b1.reshape(1, F), W2, b2.reshape(1, F), We, be.reshape(1, F))


_C = 48
_EPT = E // 16
_NFULL = _EPT // _C
_TAIL = _EPT - _NFULL * _C
_NPT = N // 16


def _sc_body(node_hbm, dist_hbm, he_hbm, src_hbm, dst_hbm, out_hbm,
             acc, src0, src1, dst0, dst1, sdst0, sdst1,
             nf0, nf1, dist0, dist1, he0, he1, tsrc, tdst,
             semld0, semld1, semg0, semg1, sems0, sems1):
    c = lax.axis_index("c")
    s = lax.axis_index("s")
    srcv = (src0, src1)
    dstv = (dst0, dst1)
    sdstv = (sdst0, sdst1)
    nfv = (nf0, nf1)
    distv = (dist0, dist1)
    hev = (he0, he1)
    semld = (semld0, semld1)
    semg = (semg0, semg1)
    sems = (sems0, sems1)
    tile_base = s * _EPT

    def _zrow(i, _):
        for j in range(FH // 16):
            nf0[i, pl.ds(j * 16, 16)] = jnp.zeros((16,), jnp.float32)
        return ()
    lax.fori_loop(0, _C, _zrow, ())
    base = s * _NPT
    for k in range(_NPT // _C):
        pltpu.sync_copy(nf0, acc.at[pl.ds(base + k * _C, _C)])
    rem = _NPT % _C
    if rem:
        pltpu.sync_copy(nf0.at[pl.ds(0, rem)],
                        acc.at[pl.ds(base + (_NPT // _C) * _C, rem)])
    plsc.subcore_barrier()

    def issue_loads(k, b):
        e0 = tile_base + k * _C
        pltpu.async_copy(src_hbm.at[pl.ds(e0, _C)], srcv[b], semld[b])
        pltpu.async_copy(dst_hbm.at[pl.ds(e0, _C)], dstv[b], semld[b])
        pltpu.async_copy(dist_hbm.at[c, pl.ds(e0, _C)], distv[b], semld[b])
        pltpu.async_copy(he_hbm.at[pl.ds(e0, _C), c], hev[b], semld[b])

    def wait_loads(b):
        pltpu.make_async_copy(src_hbm.at[pl.ds(0, _C)], srcv[b], semld[b]).wait()
        pltpu.make_async_copy(dst_hbm.at[pl.ds(0, _C)], dstv[b], semld[b]).wait()
        pltpu.make_async_copy(dist_hbm.at[c, pl.ds(0, _C)], distv[b], semld[b]).wait()
        pltpu.make_async_copy(he_hbm.at[pl.ds(0, _C), c], hev[b], semld[b]).wait()

    def start_gather(b):
        for g in range(_C // 16):
            sl = pl.ds(g * 16, 16)
            srcv[b][sl] = srcv[b][sl] * 2 + c
        pltpu.async_copy(node_hbm.at[srcv[b]], nfv[b], semg[b])

    def wait_gather(b):
        pltpu.make_async_copy(node_hbm.at[srcv[b]], nfv[b], semg[b]).wait()

    def fma_scatter(b):
        for g in range(_C // 16):
            sl = pl.ds(g * 16, 16)
            sdstv[b][sl] = dstv[b][sl]

        def _fma(i, _):
            for j in range(FH // 16):
                sl = pl.ds(j * 16, 16)
                nfv[b][i, sl] = nfv[b][i, sl] * distv[b][i, sl] + hev[b][i, sl]
            return ()
        lax.fori_loop(0, _C, _fma, ())
        pltpu.async_copy(nfv[b], acc.at[sdstv[b]], sems[b], add=True)

    def wait_scatter(b):
        pltpu.make_async_copy(nfv[b], acc.at[sdstv[b]], sems[b]).wait()

    issue_loads(0, 0)
    wait_loads(0)
    start_gather(0)
    issue_loads(1, 1)
    wait_loads(1)
    start_gather(1)
    wait_gather(0)
    fma_scatter(0)
    issue_loads(2, 0)

    def dstep(kk, b):
        o = 1 - b
        wait_loads(b)
        wait_scatter(b)
        start_gather(b)
        wait_gather(o)
        fma_scatter(o)
        e_next = tile_base + (kk + 1) * _C
        pltpu.async_copy(src_hbm.at[pl.ds(e_next, _C)], srcv[o], semld[o])
        pltpu.async_copy(dst_hbm.at[pl.ds(e_next, _C)], dstv[o], semld[o])
        pltpu.async_copy(dist_hbm.at[c, pl.ds(e_next, _C)], distv[o], semld[o])
        pltpu.async_copy(he_hbm.at[pl.ds(e_next, _C), c], hev[o], semld[o])

    def _pair(j, _):
        k = 2 * j + 2
        dstep(k, 0)
        dstep(k + 1, 1)
        return ()
    lax.fori_loop(0, (_NFULL - 4) // 2, _pair, ())

    dstep(_NFULL - 2, 0)
    wait_loads(1)
    wait_scatter(1)
    start_gather(1)
    wait_gather(0)
    fma_scatter(0)
    wait_gather(1)
    fma_scatter(1)
    wait_scatter(0)
    wait_scatter(1)

    t0 = tile_base + _NFULL * _C
    tl = pl.ds(0, _TAIL)
    pltpu.sync_copy(src_hbm.at[pl.ds(t0, _TAIL)], tsrc)
    pltpu.sync_copy(dst_hbm.at[pl.ds(t0, _TAIL)], tdst)
    pltpu.sync_copy(dist_hbm.at[c, pl.ds(t0, _TAIL)], dist0.at[tl])
    pltpu.sync_copy(he_hbm.at[pl.ds(t0, _TAIL), c], he0.at[tl])
    tsrc[pl.ds(0, 16)] = tsrc[pl.ds(0, 16)] * 2 + c
    pltpu.async_copy(node_hbm.at[tsrc], nf0.at[tl], semg0).wait()

    def _tfma(i, _):
        for j in range(FH // 16):
            sl = pl.ds(j * 16, 16)
            nf0[i, sl] = nf0[i, sl] * dist0[i, sl] + he0[i, sl]
        return ()
    lax.fori_loop(0, _TAIL, _tfma, ())
    pltpu.sync_copy(nf0.at[tl], acc.at[tdst], add=True)

    plsc.subcore_barrier()
    r0 = s * _NPT
    pltpu.sync_copy(acc.at[pl.ds(r0, _NPT)], out_hbm.at[pl.ds(r0, _NPT), c])


def _sc_message(node_flat, dist_split, he, src, dst):
    mesh = plsc.VectorSubcoreMesh(core_axis_name="c", subcore_axis_name="s")
    f = pl.kernel(
        _sc_body,
        out_type=jax.ShapeDtypeStruct((N, 2, FH), jnp.float32),
        mesh=mesh,
        scratch_types=[
            pltpu.VMEM_SHARED((N, FH), jnp.float32),
            pltpu.VMEM((_C,), jnp.int32),
            pltpu.VMEM((_C,), jnp.int32),
            pltpu.VMEM((_C,), jnp.int32),
            pltpu.VMEM((_C,), jnp.int32),
            pltpu.VMEM((_C,), jnp.int32),
            pltpu.VMEM((_C,), jnp.int32),
            pltpu.VMEM((_C, FH), jnp.float32),
            pltpu.VMEM((_C, FH), jnp.float32),
            pltpu.VMEM((_C, FH), jnp.float32),
            pltpu.VMEM((_C, FH), jnp.float32),
            pltpu.VMEM((_C, FH), jnp.float32),
            pltpu.VMEM((_C, FH), jnp.float32),
            pltpu.VMEM((_TAIL,), jnp.int32),
            pltpu.VMEM((_TAIL,), jnp.int32),
            pltpu.SemaphoreType.DMA,
            pltpu.SemaphoreType.DMA,
            pltpu.SemaphoreType.DMA,
            pltpu.SemaphoreType.DMA,
            pltpu.SemaphoreType.DMA,
            pltpu.SemaphoreType.DMA,
        ],
    )
    return f(node_flat, dist_split, he.reshape(E, 2, FH), src, dst)


def kernel(node_feats, edge_feats, expanded_dists, edge_index, W1, b1, W2, b2, We, be):
    dist_split, he = _tc_dense(expanded_dists, edge_feats, W1, b1, W2, b2, We, be)
    node_flat = node_feats.reshape(2 * N, FH)
    agg = _sc_message(node_flat, dist_split, he, edge_index[0], edge_index[1])
    return (agg.reshape(N, F), he)
